# initial kernel scaffold (unmeasured)
import jax
import jax.numpy as jnp
from jax import lax
from jax.experimental import pallas as pl
from jax.experimental.pallas import tpu as pltpu

N_DEV = 8


def kernel(x, w_mat):
    m_per, k = x.shape
    _, n_per = w_mat.shape
    half = m_per // 2
    m_tot = N_DEV * m_per

    def body(x_ref, w_ref, out_ref, cw, ccw, w_bf, amax_tx, amax_rx,
             cw_ssem, cw_rsem, ccw_ssem, ccw_rsem, ax_ssem, ax_rsem):
        my = lax.axis_index("i")
        left = lax.rem(my + (N_DEV - 1), N_DEV)
        right = lax.rem(my + 1, N_DEV)

        bar = pltpu.get_barrier_semaphore()
        for nbr in (left, right):
            pl.semaphore_signal(bar, inc=1, device_id=(nbr,),
                                device_id_type=pl.DeviceIdType.MESH)
        pl.semaphore_wait(bar, 2)

        cw[0] = x_ref[0:half, :].astype(jnp.bfloat16)
        ccw[0] = x_ref[half:m_per, :].astype(jnp.bfloat16)
        w_bf[...] = w_ref[...].astype(jnp.bfloat16)

        def ring_descs(h):
            d_cw = pltpu.make_async_remote_copy(
                src_ref=cw.at[h], dst_ref=cw.at[h + 1],
                send_sem=cw_ssem.at[h], recv_sem=cw_rsem.at[h + 1],
                device_id=(right,), device_id_type=pl.DeviceIdType.MESH)
            d_ccw = pltpu.make_async_remote_copy(
                src_ref=ccw.at[h], dst_ref=ccw.at[h + 1],
                send_sem=ccw_ssem.at[h], recv_sem=ccw_rsem.at[h + 1],
                device_id=(left,), device_id_type=pl.DeviceIdType.MESH)
            return d_cw, d_ccw

        amax = jnp.float32(0.0)

        def compute_slot(s, amax):
            o_cw = lax.rem(my - s + N_DEV, N_DEV)
            o_ccw = lax.rem(my + s, N_DEV)
            y_cw = jnp.dot(cw[s], w_bf[...],
                           preferred_element_type=jnp.float32)
            out_ref[pl.ds(o_cw * m_per, half), :] = y_cw
            amax = jnp.maximum(amax, jnp.max(jnp.abs(y_cw)))
            y_ccw = jnp.dot(ccw[s], w_bf[...],
                            preferred_element_type=jnp.float32)
            out_ref[pl.ds(o_ccw * m_per + half, half), :] = y_ccw
            return jnp.maximum(amax, jnp.max(jnp.abs(y_ccw)))

        for h in range(N_DEV - 1):
            d_cw, d_ccw = ring_descs(h)
            d_cw.start()
            d_ccw.start()
            amax = compute_slot(h, amax)
            d_cw.wait()
            d_ccw.wait()
        amax = compute_slot(N_DEV - 1, amax)

        amax_tx[...] = jnp.broadcast_to(amax, (1, 128))
        amax_rx[pl.ds(my, 1), :] = amax_tx[...]

        def ax_desc(p, slot_sem):
            return pltpu.make_async_remote_copy(
                src_ref=amax_tx, dst_ref=amax_rx.at[pl.ds(my, 1)],
                send_sem=ax_ssem.at[p], recv_sem=slot_sem,
                device_id=(p,), device_id_type=pl.DeviceIdType.MESH)

        for p in range(N_DEV):
            @pl.when(my != p)
            def _():
                ax_desc(p, ax_rsem.at[my]).start()
        for p in range(N_DEV):
            @pl.when(my != p)
            def _():
                r = pltpu.make_async_remote_copy(
                    src_ref=amax_tx, dst_ref=amax_rx.at[pl.ds(p, 1)],
                    send_sem=ax_ssem.at[p], recv_sem=ax_rsem.at[p],
                    device_id=(p,), device_id_type=pl.DeviceIdType.MESH)
                r.wait_recv()
        for p in range(N_DEV):
            @pl.when(my != p)
            def _():
                ax_desc(p, ax_rsem.at[my]).wait_send()

        g_amax = jnp.max(amax_rx[...])
        scale = g_amax / 448.0
        v = jnp.clip(out_ref[...] / scale, -448.0, 448.0)
        q = v.astype(jnp.float8_e4m3fn).astype(jnp.float32)
        out_ref[...] = q * scale

    return pl.pallas_call(
        body,
        out_shape=jax.ShapeDtypeStruct((m_tot, n_per), jnp.float32),
        in_specs=[pl.BlockSpec(memory_space=pltpu.VMEM),
                  pl.BlockSpec(memory_space=pltpu.VMEM)],
        out_specs=pl.BlockSpec(memory_space=pltpu.VMEM),
        scratch_shapes=[
            pltpu.VMEM((N_DEV, half, k), jnp.bfloat16),
            pltpu.VMEM((N_DEV, half, k), jnp.bfloat16),
            pltpu.VMEM((k, n_per), jnp.bfloat16),
            pltpu.VMEM((1, 128), jnp.float32),
            pltpu.VMEM((N_DEV, 128), jnp.float32),
            pltpu.SemaphoreType.DMA((N_DEV,)),
            pltpu.SemaphoreType.DMA((N_DEV,)),
            pltpu.SemaphoreType.DMA((N_DEV,)),
            pltpu.SemaphoreType.DMA((N_DEV,)),
            pltpu.SemaphoreType.DMA((N_DEV,)),
            pltpu.SemaphoreType.DMA((N_DEV,)),
        ],
        compiler_params=pltpu.CompilerParams(collective_id=0),
    )(x, w_mat)


# baseline (device time: 192837 ns/iter reference)
import jax
import jax.numpy as jnp
from jax import lax
from jax.experimental import pallas as pl
from jax.experimental.pallas import tpu as pltpu

N_DEV = 8


def kernel(x, w_mat):
    m_per, k = x.shape
    _, n_per = w_mat.shape
    half = m_per // 2
    m_tot = N_DEV * m_per

    def body(x_ref, w_ref, out_ref, cw, ccw, w_bf, amax_tx, amax_rx,
             cw_ssem, cw_rsem, ccw_ssem, ccw_rsem, ax_ssem, ax_rsem):
        my = lax.axis_index("i")
        left = lax.rem(my + (N_DEV - 1), N_DEV)
        right = lax.rem(my + 1, N_DEV)

        bar = pltpu.get_barrier_semaphore()
        for nbr in (left, right):
            pl.semaphore_signal(bar, inc=1, device_id=(nbr,),
                                device_id_type=pl.DeviceIdType.MESH)
        pl.semaphore_wait(bar, 2)

        cw[0] = x_ref[0:half, :].astype(jnp.bfloat16)
        ccw[0] = x_ref[half:m_per, :].astype(jnp.bfloat16)
        w_bf[...] = w_ref[...].astype(jnp.bfloat16)

        def ring_descs(h):
            d_cw = pltpu.make_async_remote_copy(
                src_ref=cw.at[h], dst_ref=cw.at[h + 1],
                send_sem=cw_ssem.at[h], recv_sem=cw_rsem.at[h + 1],
                device_id=(right,), device_id_type=pl.DeviceIdType.MESH)
            d_ccw = pltpu.make_async_remote_copy(
                src_ref=ccw.at[h], dst_ref=ccw.at[h + 1],
                send_sem=ccw_ssem.at[h], recv_sem=ccw_rsem.at[h + 1],
                device_id=(left,), device_id_type=pl.DeviceIdType.MESH)
            return d_cw, d_ccw

        amax = jnp.float32(0.0)

        def compute_slot(s, amax):
            o_cw = lax.rem(my - s + N_DEV, N_DEV)
            o_ccw = lax.rem(my + s, N_DEV)
            y_cw = jnp.dot(cw[s], w_bf[...],
                           preferred_element_type=jnp.float32)
            out_ref[pl.ds(o_cw * m_per, half), :] = y_cw
            amax = jnp.maximum(amax, jnp.max(jnp.abs(y_cw)))
            y_ccw = jnp.dot(ccw[s], w_bf[...],
                            preferred_element_type=jnp.float32)
            out_ref[pl.ds(o_ccw * m_per + half, half), :] = y_ccw
            return jnp.maximum(amax, jnp.max(jnp.abs(y_ccw)))

        for h in range(N_DEV - 1):
            d_cw, d_ccw = ring_descs(h)
            d_cw.start()
            d_ccw.start()
            amax = compute_slot(h, amax)
            d_cw.wait()
            d_ccw.wait()
        amax = compute_slot(N_DEV - 1, amax)

        amax_tx[...] = jnp.broadcast_to(amax, (1, 128))
        amax_rx[pl.ds(my, 1), :] = amax_tx[...]

        def ax_desc(p, slot_sem):
            return pltpu.make_async_remote_copy(
                src_ref=amax_tx, dst_ref=amax_rx.at[pl.ds(my, 1)],
                send_sem=ax_ssem.at[p], recv_sem=slot_sem,
                device_id=(p,), device_id_type=pl.DeviceIdType.MESH)

        for p in range(N_DEV):
            @pl.when(my != p)
            def _():
                ax_desc(p, ax_rsem.at[my]).start()
        for p in range(N_DEV):
            @pl.when(my != p)
            def _():
                r = pltpu.make_async_remote_copy(
                    src_ref=amax_tx, dst_ref=amax_rx.at[pl.ds(p, 1)],
                    send_sem=ax_ssem.at[p], recv_sem=ax_rsem.at[p],
                    device_id=(p,), device_id_type=pl.DeviceIdType.MESH)
                r.wait_recv()
        for p in range(N_DEV):
            @pl.when(my != p)
            def _():
                ax_desc(p, ax_rsem.at[my]).wait_send()

        g_amax = jnp.max(amax_rx[...])
        scale = g_amax / 448.0
        v = jnp.clip(out_ref[...] / scale, -448.0, 448.0)
        q = v.astype(jnp.float8_e4m3fn).astype(jnp.float32)
        out_ref[...] = q * scale

    return pl.pallas_call(
        body,
        out_shape=jax.ShapeDtypeStruct((m_tot, n_per), jnp.float32),
        in_specs=[pl.BlockSpec(memory_space=pltpu.VMEM),
                  pl.BlockSpec(memory_space=pltpu.VMEM)],
        out_specs=pl.BlockSpec(memory_space=pltpu.VMEM),
        scratch_shapes=[
            pltpu.VMEM((N_DEV, half, k), jnp.bfloat16),
            pltpu.VMEM((N_DEV, half, k), jnp.bfloat16),
            pltpu.VMEM((k, n_per), jnp.bfloat16),
            pltpu.VMEM((1, 128), jnp.float32),
            pltpu.VMEM((N_DEV, 128), jnp.float32),
            pltpu.SemaphoreType.DMA((N_DEV,)),
            pltpu.SemaphoreType.DMA((N_DEV,)),
            pltpu.SemaphoreType.DMA((N_DEV,)),
            pltpu.SemaphoreType.DMA((N_DEV,)),
            pltpu.SemaphoreType.DMA((N_DEV,)),
            pltpu.SemaphoreType.DMA((N_DEV,)),
        ],
        compiler_params=pltpu.CompilerParams(
            collective_id=0, vmem_limit_bytes=100 * 1024 * 1024),
    )(x, w_mat)


# device time: 188238 ns/iter; 1.0244x vs baseline; 1.0244x over previous
import jax
import jax.numpy as jnp
from jax import lax
from jax.experimental import pallas as pl
from jax.experimental.pallas import tpu as pltpu

N_DEV = 8
N_STREAM = 4


def kernel(x, w_mat):
    m_per, k = x.shape
    _, n_per = w_mat.shape
    sub = m_per // N_STREAM
    m_tot = N_DEV * m_per

    xb = x.astype(jnp.bfloat16)
    wb = w_mat.astype(jnp.bfloat16)

    def perm(v):
        return jnp.where(v >= 4, 11 - v, v)

    def body(x_ref, w_ref, out_ref, slots, amax_tx, amax_rx,
             ssems, rsems, ax_ssem, ax_rsem):
        my = lax.axis_index("i")
        r = perm(my)
        right = perm(lax.rem(r + 1, N_DEV))
        left = perm(lax.rem(r + (N_DEV - 1), N_DEV))

        bar = pltpu.get_barrier_semaphore()
        for nbr in (left, right):
            pl.semaphore_signal(bar, inc=1, device_id=(nbr,),
                                device_id_type=pl.DeviceIdType.MESH)
        pl.semaphore_wait(bar, 2)

        def hop_desc(st, h):
            tgt = right if st < 2 else left
            src = x_ref.at[pl.ds(st * sub, sub), :] if h == 0 \
                else slots.at[st, h]
            return pltpu.make_async_remote_copy(
                src_ref=src, dst_ref=slots.at[st, h + 1],
                send_sem=ssems.at[st, h], recv_sem=rsems.at[st, h + 1],
                device_id=(tgt,), device_id_type=pl.DeviceIdType.MESH)

        def recv_desc(st, h):
            return pltpu.make_async_remote_copy(
                src_ref=slots.at[st, h], dst_ref=slots.at[st, h],
                send_sem=ssems.at[st, h], recv_sem=rsems.at[st, h],
                device_id=(right,), device_id_type=pl.DeviceIdType.MESH)

        for st in range(N_STREAM):
            hop_desc(st, 0).start()

        amax = jnp.float32(0.0)

        def accum_store(y, row0, amax):
            out_ref[pl.ds(row0, y.shape[0]), :] = y
            return jnp.maximum(amax, jnp.max(jnp.abs(y)))

        y0 = jnp.dot(x_ref[...], w_ref[...],
                     preferred_element_type=jnp.float32)
        amax = accum_store(y0, my * m_per, amax)

        for h in range(1, N_DEV):
            for st in range(N_STREAM):
                recv_desc(st, h).wait_recv()
                if h < N_DEV - 1:
                    hop_desc(st, h).start()
            o_cw = perm(lax.rem(r - h + N_DEV, N_DEV))
            o_ccw = perm(lax.rem(r + h, N_DEV))
            for st in range(N_STREAM):
                o = o_cw if st < 2 else o_ccw
                y = jnp.dot(slots[st, h], w_ref[...],
                            preferred_element_type=jnp.float32)
                amax = accum_store(y, o * m_per + st * sub, amax)

        for st in range(N_STREAM):
            for h in range(N_DEV - 1):
                hop_desc(st, h).wait_send()

        amax_tx[...] = jnp.broadcast_to(amax, (1, 128))
        amax_rx[pl.ds(my, 1), :] = amax_tx[...]

        def ax_desc(p):
            return pltpu.make_async_remote_copy(
                src_ref=amax_tx, dst_ref=amax_rx.at[pl.ds(my, 1)],
                send_sem=ax_ssem.at[p], recv_sem=ax_rsem.at[my],
                device_id=(p,), device_id_type=pl.DeviceIdType.MESH)

        for p in range(N_DEV):
            @pl.when(my != p)
            def _():
                ax_desc(p).start()
        for p in range(N_DEV):
            @pl.when(my != p)
            def _():
                rd = pltpu.make_async_remote_copy(
                    src_ref=amax_tx, dst_ref=amax_rx.at[pl.ds(p, 1)],
                    send_sem=ax_ssem.at[p], recv_sem=ax_rsem.at[p],
                    device_id=(p,), device_id_type=pl.DeviceIdType.MESH)
                rd.wait_recv()
        for p in range(N_DEV):
            @pl.when(my != p)
            def _():
                ax_desc(p).wait_send()

        g_amax = jnp.max(amax_rx[...])
        scale = g_amax / 448.0
        v = jnp.clip(out_ref[...] / scale, -448.0, 448.0)
        q = v.astype(jnp.float8_e4m3fn).astype(jnp.float32)
        out_ref[...] = q * scale

    return pl.pallas_call(
        body,
        out_shape=jax.ShapeDtypeStruct((m_tot, n_per), jnp.float32),
        in_specs=[pl.BlockSpec(memory_space=pltpu.VMEM),
                  pl.BlockSpec(memory_space=pltpu.VMEM)],
        out_specs=pl.BlockSpec(memory_space=pltpu.VMEM),
        scratch_shapes=[
            pltpu.VMEM((N_STREAM, N_DEV, sub, k), jnp.bfloat16),
            pltpu.VMEM((1, 128), jnp.float32),
            pltpu.VMEM((N_DEV, 128), jnp.float32),
            pltpu.SemaphoreType.DMA((N_STREAM, N_DEV)),
            pltpu.SemaphoreType.DMA((N_STREAM, N_DEV)),
            pltpu.SemaphoreType.DMA((N_DEV,)),
            pltpu.SemaphoreType.DMA((N_DEV,)),
        ],
        compiler_params=pltpu.CompilerParams(
            collective_id=0, vmem_limit_bytes=100 * 1024 * 1024),
    )(xb, wb)


# device time: 187505 ns/iter; 1.0284x vs baseline; 1.0039x over previous
import jax
import jax.numpy as jnp
from jax import lax
from jax.experimental import pallas as pl
from jax.experimental.pallas import tpu as pltpu

N_DEV = 8
N_STREAM = 4


def kernel(x, w_mat):
    m_per, k = x.shape
    _, n_per = w_mat.shape
    sub = m_per // N_STREAM
    m_tot = N_DEV * m_per

    xb = x.astype(jnp.bfloat16)
    wb = w_mat.astype(jnp.bfloat16)

    def perm(v):
        return jnp.where(v >= 4, 11 - v, v)

    def body(x_ref, w_ref, out_ref, slots, amax_tx, amax_rx,
             ssems, rsems, ax_ssem, ax_rsem):
        my = lax.axis_index("i")
        r = perm(my)
        right = perm(lax.rem(r + 1, N_DEV))
        left = perm(lax.rem(r + (N_DEV - 1), N_DEV))

        bar = pltpu.get_barrier_semaphore()
        for nbr in (left, right):
            pl.semaphore_signal(bar, inc=1, device_id=(nbr,),
                                device_id_type=pl.DeviceIdType.MESH)
        pl.semaphore_wait(bar, 2)

        def hop_desc(st, h):
            tgt = right if st < 2 else left
            src = x_ref.at[pl.ds(st * sub, sub), :] if h == 0 \
                else slots.at[st, h]
            return pltpu.make_async_remote_copy(
                src_ref=src, dst_ref=slots.at[st, h + 1],
                send_sem=ssems.at[st, h], recv_sem=rsems.at[st, h + 1],
                device_id=(tgt,), device_id_type=pl.DeviceIdType.MESH)

        def recv_desc(st, h):
            return pltpu.make_async_remote_copy(
                src_ref=slots.at[st, h], dst_ref=slots.at[st, h],
                send_sem=ssems.at[st, h], recv_sem=rsems.at[st, h],
                device_id=(right,), device_id_type=pl.DeviceIdType.MESH)

        for st in range(N_STREAM):
            hop_desc(st, 0).start()

        amax = jnp.float32(0.0)

        def accum_store(y, row0, amax):
            out_ref[pl.ds(row0, y.shape[0]), :] = y
            return jnp.maximum(amax, jnp.max(jnp.abs(y)))

        y0 = jnp.dot(x_ref[...], w_ref[...],
                     preferred_element_type=jnp.float32)
        amax = accum_store(y0, my * m_per, amax)

        for h in range(1, N_DEV):
            for st in range(N_STREAM):
                recv_desc(st, h).wait_recv()
                if h < N_DEV - 1:
                    hop_desc(st, h).start()
            o_cw = perm(lax.rem(r - h + N_DEV, N_DEV))
            o_ccw = perm(lax.rem(r + h, N_DEV))
            for st in range(N_STREAM):
                o = o_cw if st < 2 else o_ccw
                y = jnp.dot(slots[st, h], w_ref[...],
                            preferred_element_type=jnp.float32)
                amax = accum_store(y, o * m_per + st * sub, amax)

        for st in range(N_STREAM):
            for h in range(N_DEV - 1):
                hop_desc(st, h).wait_send()

        amax_tx[...] = jnp.broadcast_to(amax, (1, 128))
        amax_rx[pl.ds(my, 1), :] = amax_tx[...]

        def ax_desc(p):
            return pltpu.make_async_remote_copy(
                src_ref=amax_tx, dst_ref=amax_rx.at[pl.ds(my, 1)],
                send_sem=ax_ssem.at[p], recv_sem=ax_rsem.at[my],
                device_id=(p,), device_id_type=pl.DeviceIdType.MESH)

        for p in range(N_DEV):
            @pl.when(my != p)
            def _():
                ax_desc(p).start()
        for p in range(N_DEV):
            @pl.when(my != p)
            def _():
                rd = pltpu.make_async_remote_copy(
                    src_ref=amax_tx, dst_ref=amax_rx.at[pl.ds(p, 1)],
                    send_sem=ax_ssem.at[p], recv_sem=ax_rsem.at[p],
                    device_id=(p,), device_id_type=pl.DeviceIdType.MESH)
                rd.wait_recv()
        for p in range(N_DEV):
            @pl.when(my != p)
            def _():
                ax_desc(p).wait_send()

        g_amax = jnp.max(amax_rx[...])
        if True:
            out_ref[0:1, 0:128] = jnp.broadcast_to(g_amax, (1, 128))
        else:
            scale = g_amax / 448.0
            inv = 448.0 / g_amax
            v = jnp.clip(out_ref[...] * inv, -448.0, 448.0)
            q = v.astype(jnp.float8_e4m3fn).astype(jnp.float32)
            out_ref[...] = q * scale

    return pl.pallas_call(
        body,
        out_shape=jax.ShapeDtypeStruct((m_tot, n_per), jnp.float32),
        in_specs=[pl.BlockSpec(memory_space=pltpu.VMEM),
                  pl.BlockSpec(memory_space=pltpu.VMEM)],
        out_specs=pl.BlockSpec(memory_space=pltpu.VMEM),
        scratch_shapes=[
            pltpu.VMEM((N_STREAM, N_DEV, sub, k), jnp.bfloat16),
            pltpu.VMEM((1, 128), jnp.float32),
            pltpu.VMEM((N_DEV, 128), jnp.float32),
            pltpu.SemaphoreType.DMA((N_STREAM, N_DEV)),
            pltpu.SemaphoreType.DMA((N_STREAM, N_DEV)),
            pltpu.SemaphoreType.DMA((N_DEV,)),
            pltpu.SemaphoreType.DMA((N_DEV,)),
        ],
        compiler_params=pltpu.CompilerParams(
            collective_id=0, vmem_limit_bytes=100 * 1024 * 1024),
    )(xb, wb)


# device time: 185854 ns/iter; 1.0376x vs baseline; 1.0089x over previous
import jax
import jax.numpy as jnp
from jax import lax
from jax.experimental import pallas as pl
from jax.experimental.pallas import tpu as pltpu

N_DEV = 8
N_STREAM = 4


def kernel(x, w_mat):
    m_per, k = x.shape
    _, n_per = w_mat.shape
    sub = m_per // N_STREAM
    m_tot = N_DEV * m_per

    xb = x.astype(jnp.bfloat16)
    wb = w_mat.astype(jnp.bfloat16)

    def perm(v):
        return jnp.where(v >= 4, 11 - v, v)

    def body(x_ref, w_ref, out_ref, slots, amax_tx, amax_rx,
             ssems, rsems, ax_ssem, ax_rsem):
        my = lax.axis_index("i")
        r = perm(my)
        right = perm(lax.rem(r + 1, N_DEV))
        left = perm(lax.rem(r + (N_DEV - 1), N_DEV))

        bar = pltpu.get_barrier_semaphore()
        for nbr in (left, right):
            pl.semaphore_signal(bar, inc=1, device_id=(nbr,),
                                device_id_type=pl.DeviceIdType.MESH)
        pl.semaphore_wait(bar, 2)

        def hop_desc(st, h):
            tgt = right if st < 2 else left
            src = x_ref.at[pl.ds(st * sub, sub), :] if h == 0 \
                else slots.at[st, h]
            return pltpu.make_async_remote_copy(
                src_ref=src, dst_ref=slots.at[st, h + 1],
                send_sem=ssems.at[st, h], recv_sem=rsems.at[st, h + 1],
                device_id=(tgt,), device_id_type=pl.DeviceIdType.MESH)

        def recv_desc(st, h):
            return pltpu.make_async_remote_copy(
                src_ref=slots.at[st, h], dst_ref=slots.at[st, h],
                send_sem=ssems.at[st, h], recv_sem=rsems.at[st, h],
                device_id=(right,), device_id_type=pl.DeviceIdType.MESH)

        for st in range(N_STREAM):
            hop_desc(st, 0).start()

        amax = jnp.float32(0.0)

        def accum_store(y, row0, amax):
            out_ref[pl.ds(row0, y.shape[0]), :] = y
            return jnp.maximum(amax, jnp.max(jnp.abs(y)))

        y0 = jnp.dot(x_ref[...], w_ref[...],
                     preferred_element_type=jnp.float32)
        amax = accum_store(y0, my * m_per, amax)

        for h in range(1, N_DEV):
            for st in range(N_STREAM):
                recv_desc(st, h).wait_recv()
                if h < N_DEV - 1:
                    hop_desc(st, h).start()
            if False:
                o_cw = perm(lax.rem(r - h + N_DEV, N_DEV))
                o_ccw = perm(lax.rem(r + h, N_DEV))
                for st in range(N_STREAM):
                    o = o_cw if st < 2 else o_ccw
                    y = jnp.dot(slots[st, h], w_ref[...],
                                preferred_element_type=jnp.float32)
                    amax = accum_store(y, o * m_per + st * sub, amax)

        for st in range(N_STREAM):
            for h in range(N_DEV - 1):
                hop_desc(st, h).wait_send()

        amax_tx[...] = jnp.broadcast_to(amax, (1, 128))
        amax_rx[pl.ds(my, 1), :] = amax_tx[...]

        def ax_desc(p):
            return pltpu.make_async_remote_copy(
                src_ref=amax_tx, dst_ref=amax_rx.at[pl.ds(my, 1)],
                send_sem=ax_ssem.at[p], recv_sem=ax_rsem.at[my],
                device_id=(p,), device_id_type=pl.DeviceIdType.MESH)

        for p in range(N_DEV):
            @pl.when(my != p)
            def _():
                ax_desc(p).start()
        for p in range(N_DEV):
            @pl.when(my != p)
            def _():
                rd = pltpu.make_async_remote_copy(
                    src_ref=amax_tx, dst_ref=amax_rx.at[pl.ds(p, 1)],
                    send_sem=ax_ssem.at[p], recv_sem=ax_rsem.at[p],
                    device_id=(p,), device_id_type=pl.DeviceIdType.MESH)
                rd.wait_recv()
        for p in range(N_DEV):
            @pl.when(my != p)
            def _():
                ax_desc(p).wait_send()

        g_amax = jnp.max(amax_rx[...])
        if True:
            out_ref[0:1, 0:128] = jnp.broadcast_to(g_amax, (1, 128))
        else:
            scale = g_amax / 448.0
            inv = 448.0 / g_amax
            v = jnp.clip(out_ref[...] * inv, -448.0, 448.0)
            q = v.astype(jnp.float8_e4m3fn).astype(jnp.float32)
            out_ref[...] = q * scale

    return pl.pallas_call(
        body,
        out_shape=jax.ShapeDtypeStruct((m_tot, n_per), jnp.float32),
        in_specs=[pl.BlockSpec(memory_space=pltpu.VMEM),
                  pl.BlockSpec(memory_space=pltpu.VMEM)],
        out_specs=pl.BlockSpec(memory_space=pltpu.VMEM),
        scratch_shapes=[
            pltpu.VMEM((N_STREAM, N_DEV, sub, k), jnp.bfloat16),
            pltpu.VMEM((1, 128), jnp.float32),
            pltpu.VMEM((N_DEV, 128), jnp.float32),
            pltpu.SemaphoreType.DMA((N_STREAM, N_DEV)),
            pltpu.SemaphoreType.DMA((N_STREAM, N_DEV)),
            pltpu.SemaphoreType.DMA((N_DEV,)),
            pltpu.SemaphoreType.DMA((N_DEV,)),
        ],
        compiler_params=pltpu.CompilerParams(
            collective_id=0, vmem_limit_bytes=100 * 1024 * 1024),
    )(xb, wb)


# device time: 145824 ns/iter; 1.3224x vs baseline; 1.2745x over previous
import jax
import jax.numpy as jnp
from jax import lax
from jax.experimental import pallas as pl
from jax.experimental.pallas import tpu as pltpu

N_DEV = 8


def kernel(x, w_mat):
    m_per, k = x.shape
    _, n_per = w_mat.shape
    half = m_per // 2
    m_tot = N_DEV * m_per

    xb = x.astype(jnp.bfloat16)
    wb = w_mat.astype(jnp.bfloat16)

    def perm(v):
        return jnp.where(v >= 4, 11 - v, v)

    def body(x_ref, w_ref, out_ref, xg, amax_tx, amax_rx,
             cw_s, cw_r, ccw_s, ccw_r, ch_s, ch_r, ax_s, ax_r):
        my = lax.axis_index("i")
        r = perm(my)
        even = lax.rem(r, 2) == 0
        right = perm(lax.rem(r + 1, N_DEV))
        left = perm(lax.rem(r + (N_DEV - 1), N_DEV))
        p_ring = lax.rem(jnp.where(even, r + 3, r + 5), N_DEV)
        chord = perm(p_ring)

        def row0(d):
            return perm(lax.rem(r + d + N_DEV, N_DEV)) * m_per

        bar = pltpu.get_barrier_semaphore()
        for nbr in (left, right, chord):
            pl.semaphore_signal(bar, inc=1, device_id=(nbr,),
                                device_id_type=pl.DeviceIdType.MESH)
        pl.semaphore_wait(bar, 3)

        def send(src, dst_row, n_rows, tgt, ssem, rsem):
            return pltpu.make_async_remote_copy(
                src_ref=src, dst_ref=xg.at[pl.ds(dst_row, n_rows), :],
                send_sem=ssem, recv_sem=rsem, device_id=(tgt,),
                device_id_type=pl.DeviceIdType.MESH)

        def fwd(src_row, n_rows, tgt, ssem, rsem):
            return send(xg.at[pl.ds(src_row, n_rows), :], src_row,
                        n_rows, tgt, ssem, rsem)

        def recv(dst_row, n_rows, rsem):
            return send(xg.at[pl.ds(dst_row, n_rows), :], dst_row,
                        n_rows, right, cw_s.at[0], rsem)

        my_row = my * m_per
        send(x_ref, my_row, m_per, right, cw_s.at[0], cw_r.at[0]).start()
        send(x_ref, my_row, m_per, left, ccw_s.at[0], ccw_r.at[0]).start()
        send(x_ref, my_row, m_per, chord, ch_s.at[0], ch_r.at[0]).start()

        amax = jnp.float32(0.0)

        def gemm(src, row, amax):
            y = jnp.dot(src, w_ref[...], preferred_element_type=jnp.float32)
            out_ref[pl.ds(row, src.shape[0]), :] = y
            return jnp.maximum(amax, jnp.max(jnp.abs(y)))

        amax = gemm(x_ref[...], my_row, amax)

        recv(row0(-1), m_per, cw_r.at[0]).wait_recv()
        fwd(row0(-1), m_per, right, cw_s.at[1], cw_r.at[1]).start()

        @pl.when(even)
        def _():
            fwd(row0(-1), m_per, chord, ch_s.at[1], ch_r.at[1]).start()

        recv(row0(1), m_per, ccw_r.at[0]).wait_recv()
        fwd(row0(1), m_per, left, ccw_s.at[1], ccw_r.at[1]).start()

        @pl.when(jnp.logical_not(even))
        def _():
            fwd(row0(1), m_per, chord, ch_s.at[1], ch_r.at[1]).start()

        k1_row = perm(p_ring) * m_per
        recv(k1_row, m_per, ch_r.at[0]).wait_recv()

        amax = gemm(xg[pl.ds(row0(-1), m_per), :], row0(-1), amax)
        amax = gemm(xg[pl.ds(row0(1), m_per), :], row0(1), amax)
        amax = gemm(xg[pl.ds(k1_row, m_per), :], k1_row, amax)

        recv(row0(-2), m_per, cw_r.at[1]).wait_recv()

        @pl.when(even)
        def _():
            fwd(row0(-2), half, chord, ch_s.at[2], ch_r.at[2]).start()

        @pl.when(jnp.logical_not(even))
        def _():
            fwd(row0(-2), half, right, cw_s.at[2], cw_r.at[2]).start()

        recv(row0(2), m_per, ccw_r.at[1]).wait_recv()

        @pl.when(even)
        def _():
            fwd(row0(2) + half, half, left, ccw_s.at[2], ccw_r.at[2]).start()

        @pl.when(jnp.logical_not(even))
        def _():
            fwd(row0(2) + half, half, chord, ch_s.at[2], ch_r.at[2]).start()

        recv(row0(4), m_per, ch_r.at[1]).wait_recv()

        amax = gemm(xg[pl.ds(row0(-2), m_per), :], row0(-2), amax)
        amax = gemm(xg[pl.ds(row0(2), m_per), :], row0(2), amax)
        amax = gemm(xg[pl.ds(row0(4), m_per), :], row0(4), amax)

        s_row = jnp.where(even, row0(5), row0(3))

        @pl.when(even)
        def _():
            recv(s_row, half, cw_r.at[2]).wait_recv()
            recv(s_row + half, half, ch_r.at[2]).wait_recv()

        @pl.when(jnp.logical_not(even))
        def _():
            recv(s_row, half, ch_r.at[2]).wait_recv()
            recv(s_row + half, half, ccw_r.at[2]).wait_recv()

        amax = gemm(xg[pl.ds(s_row, m_per), :], s_row, amax)

        send(x_ref, my_row, m_per, right, cw_s.at[0], cw_r.at[0]).wait_send()
        send(x_ref, my_row, m_per, left, ccw_s.at[0], ccw_r.at[0]).wait_send()
        send(x_ref, my_row, m_per, chord, ch_s.at[0], ch_r.at[0]).wait_send()
        fwd(row0(-1), m_per, right, cw_s.at[1], cw_r.at[1]).wait_send()
        fwd(row0(1), m_per, left, ccw_s.at[1], ccw_r.at[1]).wait_send()

        @pl.when(even)
        def _():
            fwd(row0(-1), m_per, chord, ch_s.at[1], ch_r.at[1]).wait_send()
            fwd(row0(-2), half, chord, ch_s.at[2], ch_r.at[2]).wait_send()
            fwd(row0(2) + half, half, left, ccw_s.at[2],
                ccw_r.at[2]).wait_send()

        @pl.when(jnp.logical_not(even))
        def _():
            fwd(row0(1), m_per, chord, ch_s.at[1], ch_r.at[1]).wait_send()
            fwd(row0(-2), half, right, cw_s.at[2], cw_r.at[2]).wait_send()
            fwd(row0(2) + half, half, chord, ch_s.at[2],
                ch_r.at[2]).wait_send()

        amax_tx[...] = jnp.broadcast_to(amax, (1, 128))
        amax_rx[pl.ds(my, 1), :] = amax_tx[...]

        def ax_desc(p):
            return pltpu.make_async_remote_copy(
                src_ref=amax_tx, dst_ref=amax_rx.at[pl.ds(my, 1)],
                send_sem=ax_s.at[p], recv_sem=ax_r.at[my],
                device_id=(p,), device_id_type=pl.DeviceIdType.MESH)

        for p in range(N_DEV):
            @pl.when(my != p)
            def _():
                ax_desc(p).start()
        for p in range(N_DEV):
            @pl.when(my != p)
            def _():
                rd = pltpu.make_async_remote_copy(
                    src_ref=amax_tx, dst_ref=amax_rx.at[pl.ds(p, 1)],
                    send_sem=ax_s.at[p], recv_sem=ax_r.at[p],
                    device_id=(p,), device_id_type=pl.DeviceIdType.MESH)
                rd.wait_recv()
        for p in range(N_DEV):
            @pl.when(my != p)
            def _():
                ax_desc(p).wait_send()

        g_amax = jnp.max(amax_rx[...])
        scale = g_amax / 448.0
        inv = 448.0 / g_amax
        v = jnp.clip(out_ref[...] * inv, -448.0, 448.0)
        q = v.astype(jnp.float8_e4m3fn).astype(jnp.float32)
        out_ref[...] = q * scale

    return pl.pallas_call(
        body,
        out_shape=jax.ShapeDtypeStruct((m_tot, n_per), jnp.float32),
        in_specs=[pl.BlockSpec(memory_space=pltpu.VMEM),
                  pl.BlockSpec(memory_space=pltpu.VMEM)],
        out_specs=pl.BlockSpec(memory_space=pltpu.VMEM),
        scratch_shapes=[
            pltpu.VMEM((m_tot, k), jnp.bfloat16),
            pltpu.VMEM((1, 128), jnp.float32),
            pltpu.VMEM((N_DEV, 128), jnp.float32),
            pltpu.SemaphoreType.DMA((3,)),
            pltpu.SemaphoreType.DMA((3,)),
            pltpu.SemaphoreType.DMA((3,)),
            pltpu.SemaphoreType.DMA((3,)),
            pltpu.SemaphoreType.DMA((3,)),
            pltpu.SemaphoreType.DMA((3,)),
            pltpu.SemaphoreType.DMA((N_DEV,)),
            pltpu.SemaphoreType.DMA((N_DEV,)),
        ],
        compiler_params=pltpu.CompilerParams(
            collective_id=0, vmem_limit_bytes=100 * 1024 * 1024),
    )(xb, wb)


# device time: 139214 ns/iter; 1.3852x vs baseline; 1.0475x over previous
import jax
import jax.numpy as jnp
from jax import lax
from jax.experimental import pallas as pl
from jax.experimental.pallas import tpu as pltpu

N_DEV = 8


def kernel(x, w_mat):
    m_per, k = x.shape
    _, n_per = w_mat.shape
    half = m_per // 2
    m_tot = N_DEV * m_per

    def perm(v):
        return jnp.where(v >= 4, 11 - v, v)

    def body(x_ref, w_ref, out_ref, xg, w_bf, amax_tx, amax_rx,
             cw_s, cw_r, ccw_s, ccw_r, ch_s, ch_r, ax_s, ax_r):
        my = lax.axis_index("i")
        r = perm(my)
        even = lax.rem(r, 2) == 0
        right = perm(lax.rem(r + 1, N_DEV))
        left = perm(lax.rem(r + (N_DEV - 1), N_DEV))
        p_ring = lax.rem(jnp.where(even, r + 3, r + 5), N_DEV)
        chord = perm(p_ring)

        def row0(d):
            return perm(lax.rem(r + d + N_DEV, N_DEV)) * m_per

        bar = pltpu.get_barrier_semaphore()
        for nbr in (left, right, chord):
            pl.semaphore_signal(bar, inc=1, device_id=(nbr,),
                                device_id_type=pl.DeviceIdType.MESH)
        pl.semaphore_wait(bar, 3)

        def send(src, dst_row, n_rows, tgt, ssem, rsem):
            return pltpu.make_async_remote_copy(
                src_ref=src, dst_ref=xg.at[pl.ds(dst_row, n_rows), :],
                send_sem=ssem, recv_sem=rsem, device_id=(tgt,),
                device_id_type=pl.DeviceIdType.MESH)

        def fwd(src_row, n_rows, tgt, ssem, rsem):
            return send(xg.at[pl.ds(src_row, n_rows), :], src_row,
                        n_rows, tgt, ssem, rsem)

        def recv(dst_row, n_rows, rsem):
            return send(xg.at[pl.ds(dst_row, n_rows), :], dst_row,
                        n_rows, right, cw_s.at[0], rsem)

        my_row = my * m_per
        own = xg.at[pl.ds(my_row, m_per), :]
        xg[pl.ds(my_row, m_per), :] = x_ref[...].astype(jnp.bfloat16)
        send(own, my_row, m_per, right, cw_s.at[0], cw_r.at[0]).start()
        send(own, my_row, m_per, left, ccw_s.at[0], ccw_r.at[0]).start()
        send(own, my_row, m_per, chord, ch_s.at[0], ch_r.at[0]).start()
        w_bf[...] = w_ref[...].astype(jnp.bfloat16)

        amax = jnp.float32(0.0)

        def gemm(src, row, amax):
            y = jnp.dot(src, w_bf[...], preferred_element_type=jnp.float32)
            out_ref[pl.ds(row, src.shape[0]), :] = y
            return jnp.maximum(amax, jnp.max(jnp.abs(y)))

        amax = gemm(xg[pl.ds(my_row, m_per), :], my_row, amax)

        recv(row0(-1), m_per, cw_r.at[0]).wait_recv()
        fwd(row0(-1), m_per, right, cw_s.at[1], cw_r.at[1]).start()

        @pl.when(even)
        def _():
            fwd(row0(-1), m_per, chord, ch_s.at[1], ch_r.at[1]).start()

        recv(row0(1), m_per, ccw_r.at[0]).wait_recv()
        fwd(row0(1), m_per, left, ccw_s.at[1], ccw_r.at[1]).start()

        @pl.when(jnp.logical_not(even))
        def _():
            fwd(row0(1), m_per, chord, ch_s.at[1], ch_r.at[1]).start()

        k1_row = perm(p_ring) * m_per
        recv(k1_row, m_per, ch_r.at[0]).wait_recv()

        amax = gemm(xg[pl.ds(row0(-1), m_per), :], row0(-1), amax)
        amax = gemm(xg[pl.ds(row0(1), m_per), :], row0(1), amax)
        amax = gemm(xg[pl.ds(k1_row, m_per), :], k1_row, amax)

        recv(row0(-2), m_per, cw_r.at[1]).wait_recv()

        @pl.when(even)
        def _():
            fwd(row0(-2), half, chord, ch_s.at[2], ch_r.at[2]).start()

        @pl.when(jnp.logical_not(even))
        def _():
            fwd(row0(-2), half, right, cw_s.at[2], cw_r.at[2]).start()

        recv(row0(2), m_per, ccw_r.at[1]).wait_recv()

        @pl.when(even)
        def _():
            fwd(row0(2) + half, half, left, ccw_s.at[2], ccw_r.at[2]).start()

        @pl.when(jnp.logical_not(even))
        def _():
            fwd(row0(2) + half, half, chord, ch_s.at[2], ch_r.at[2]).start()

        recv(row0(4), m_per, ch_r.at[1]).wait_recv()

        amax = gemm(xg[pl.ds(row0(-2), m_per), :], row0(-2), amax)
        amax = gemm(xg[pl.ds(row0(2), m_per), :], row0(2), amax)
        amax = gemm(xg[pl.ds(row0(4), m_per), :], row0(4), amax)

        s_row = jnp.where(even, row0(5), row0(3))

        @pl.when(even)
        def _():
            recv(s_row, half, cw_r.at[2]).wait_recv()
            recv(s_row + half, half, ch_r.at[2]).wait_recv()

        @pl.when(jnp.logical_not(even))
        def _():
            recv(s_row, half, ch_r.at[2]).wait_recv()
            recv(s_row + half, half, ccw_r.at[2]).wait_recv()

        amax = gemm(xg[pl.ds(s_row, m_per), :], s_row, amax)

        send(own, my_row, m_per, right, cw_s.at[0], cw_r.at[0]).wait_send()
        send(own, my_row, m_per, left, ccw_s.at[0], ccw_r.at[0]).wait_send()
        send(own, my_row, m_per, chord, ch_s.at[0], ch_r.at[0]).wait_send()
        fwd(row0(-1), m_per, right, cw_s.at[1], cw_r.at[1]).wait_send()
        fwd(row0(1), m_per, left, ccw_s.at[1], ccw_r.at[1]).wait_send()

        @pl.when(even)
        def _():
            fwd(row0(-1), m_per, chord, ch_s.at[1], ch_r.at[1]).wait_send()
            fwd(row0(-2), half, chord, ch_s.at[2], ch_r.at[2]).wait_send()
            fwd(row0(2) + half, half, left, ccw_s.at[2],
                ccw_r.at[2]).wait_send()

        @pl.when(jnp.logical_not(even))
        def _():
            fwd(row0(1), m_per, chord, ch_s.at[1], ch_r.at[1]).wait_send()
            fwd(row0(-2), half, right, cw_s.at[2], cw_r.at[2]).wait_send()
            fwd(row0(2) + half, half, chord, ch_s.at[2],
                ch_r.at[2]).wait_send()

        amax_tx[...] = jnp.broadcast_to(amax, (1, 128))
        amax_rx[pl.ds(my, 1), :] = amax_tx[...]

        def ax_desc(p):
            return pltpu.make_async_remote_copy(
                src_ref=amax_tx, dst_ref=amax_rx.at[pl.ds(my, 1)],
                send_sem=ax_s.at[p], recv_sem=ax_r.at[my],
                device_id=(p,), device_id_type=pl.DeviceIdType.MESH)

        for p in range(N_DEV):
            @pl.when(my != p)
            def _():
                ax_desc(p).start()
        for p in range(N_DEV):
            @pl.when(my != p)
            def _():
                rd = pltpu.make_async_remote_copy(
                    src_ref=amax_tx, dst_ref=amax_rx.at[pl.ds(p, 1)],
                    send_sem=ax_s.at[p], recv_sem=ax_r.at[p],
                    device_id=(p,), device_id_type=pl.DeviceIdType.MESH)
                rd.wait_recv()
        for p in range(N_DEV):
            @pl.when(my != p)
            def _():
                ax_desc(p).wait_send()

        g_amax = jnp.max(amax_rx[...])
        scale = g_amax / 448.0
        inv = 448.0 / g_amax
        v = jnp.clip(out_ref[...] * inv, -448.0, 448.0)
        q = v.astype(jnp.float8_e4m3fn).astype(jnp.float32)
        out_ref[...] = q * scale

    return pl.pallas_call(
        body,
        out_shape=jax.ShapeDtypeStruct((m_tot, n_per), jnp.float32),
        in_specs=[pl.BlockSpec(memory_space=pltpu.VMEM),
                  pl.BlockSpec(memory_space=pltpu.VMEM)],
        out_specs=pl.BlockSpec(memory_space=pltpu.VMEM),
        scratch_shapes=[
            pltpu.VMEM((m_tot, k), jnp.bfloat16),
            pltpu.VMEM((k, n_per), jnp.bfloat16),
            pltpu.VMEM((1, 128), jnp.float32),
            pltpu.VMEM((N_DEV, 128), jnp.float32),
            pltpu.SemaphoreType.DMA((3,)),
            pltpu.SemaphoreType.DMA((3,)),
            pltpu.SemaphoreType.DMA((3,)),
            pltpu.SemaphoreType.DMA((3,)),
            pltpu.SemaphoreType.DMA((3,)),
            pltpu.SemaphoreType.DMA((3,)),
            pltpu.SemaphoreType.DMA((N_DEV,)),
            pltpu.SemaphoreType.DMA((N_DEV,)),
        ],
        compiler_params=pltpu.CompilerParams(
            collective_id=0, vmem_limit_bytes=100 * 1024 * 1024),
    )(x, w_mat)
